# Initial kernel scaffold; baseline (speedup 1.0000x reference)
import jax
import jax.numpy as jnp
from jax.experimental import pallas as pl


def kernel(x, edge_index, W1, b1, gamma, beta, W2, b2):
    raise NotImplementedError("write your pallas kernel here")


# speedup vs baseline: 19.9180x; 19.9180x over previous
import dataclasses
import functools

import jax
import jax.numpy as jnp
from jax import lax
from jax.experimental import pallas as pl
from jax.experimental.pallas import tpu as pltpu
from jax.experimental.pallas import tpu_sc as plsc

_N = 10000
_E = 320000
_D = 128
_NTILES = 32
_EPT = _E // _NTILES
_CH = 80
_NCH = _EPT // _CH
_NP = 10240
_RPT = _NP // 16

_mesh = plsc.VectorSubcoreMesh(core_axis_name="c", subcore_axis_name="s")

_sc_params = pltpu.CompilerParams()
if "needs_layout_passes" in pltpu.CompilerParams.__dataclass_fields__:
    _sc_params = dataclasses.replace(_sc_params, needs_layout_passes=False)


@functools.partial(
    pl.kernel,
    out_type=jax.ShapeDtypeStruct((_NTILES, _NP), jnp.float32),
    mesh=_mesh,
    scratch_types=[
        pltpu.VMEM((_EPT // 16, 16), jnp.int32),
        pltpu.VMEM((_NP,), jnp.float32),
    ],
    compiler_params=_sc_params,
)
def _sc_deg(dst_hbm, out_hbm, dst_v, hist_v):
    c = lax.axis_index("c")
    s = lax.axis_index("s")
    w = c * 16 + s
    pltpu.sync_copy(dst_hbm.at[w], dst_v)
    zero16 = jnp.zeros((16,), jnp.float32)
    one16 = jnp.full((16,), 1.0, jnp.float32)

    @pl.loop(0, _NP // 16)
    def _(i):
        hist_v[pl.ds(pl.multiple_of(i * 16, 16), 16)] = zero16

    @pl.loop(0, _EPT // 16)
    def _(i):
        plsc.addupdate_scatter(hist_v, [dst_v[i]], one16)

    pltpu.sync_copy(hist_v, out_hbm.at[w])


@functools.partial(
    pl.kernel,
    out_type=jax.ShapeDtypeStruct((2, _NP, _D), jnp.float32),
    mesh=_mesh,
    scratch_types=[
        pltpu.VMEM((_NCH, _CH), jnp.int32),
        pltpu.VMEM((_NCH, _CH), jnp.int32),
        pltpu.VMEM((_CH, _D), jnp.float32),
        pltpu.VMEM_SHARED((_NP, _D), jnp.float32),
        pltpu.SemaphoreType.DMA,
    ],
)
def _sc_edge(g_hbm, src_hbm, dst_hbm, zeros_hbm, out_hbm,
             src_v, dst_v, rows_v, acc_sh, sem):
    c = lax.axis_index("c")
    s = lax.axis_index("s")
    w = c * 16 + s
    pltpu.sync_copy(src_hbm.at[w], src_v)
    pltpu.sync_copy(dst_hbm.at[w], dst_v)
    row0 = s * _RPT
    pltpu.sync_copy(zeros_hbm.at[pl.ds(row0, _RPT)], acc_sh.at[pl.ds(row0, _RPT)])
    plsc.subcore_barrier()

    @pl.loop(0, _NCH)
    def _(j):
        pltpu.async_copy(g_hbm.at[src_v.at[j]], rows_v, sem).wait()
        pltpu.sync_copy(rows_v, acc_sh.at[dst_v.at[j]], add=True)

    plsc.subcore_barrier()
    pltpu.sync_copy(acc_sh.at[pl.ds(row0, _RPT)], out_hbm.at[c, pl.ds(row0, _RPT)])


def _tc_h_body(x_ref, w_ref, h_ref):
    h_ref[...] = jnp.dot(x_ref[...], w_ref[...],
                         preferred_element_type=jnp.float32)


_tc_h = pl.pallas_call(
    _tc_h_body,
    out_shape=jax.ShapeDtypeStruct((_N, _D), jnp.float32),
)


def _tc_scale_body(h_ref, dr_ref, g_ref, dinv_ref):
    drt = jnp.transpose(dr_ref[...])
    deg = jnp.sum(drt[:_N], axis=1, keepdims=True) + 1.0
    dinv = lax.rsqrt(deg)
    dinv_ref[...] = dinv
    g_ref[...] = h_ref[...] * dinv


_tc_scale = pl.pallas_call(
    _tc_scale_body,
    out_shape=[
        jax.ShapeDtypeStruct((_N, _D), jnp.float32),
        jax.ShapeDtypeStruct((_N, 1), jnp.float32),
    ],
)


def _tc_mid_body(acc_ref, g_ref, dinv_ref, b_ref, gam_ref, bet_ref, w_ref,
                 out_ref):
    dinv = dinv_ref[...]
    z = dinv * (acc_ref[0, :_N] + acc_ref[1, :_N] + g_ref[...]) + b_ref[...]
    mean = jnp.mean(z, axis=0, keepdims=True)
    var = jnp.mean((z - mean) ** 2, axis=0, keepdims=True)
    z = (z - mean) * lax.rsqrt(var + 1e-5) * gam_ref[...] + bet_ref[...]
    z = jnp.maximum(z, 0.0)
    h2 = jnp.dot(z, w_ref[...], preferred_element_type=jnp.float32)
    out_ref[...] = h2 * dinv


_tc_mid = pl.pallas_call(
    _tc_mid_body,
    out_shape=jax.ShapeDtypeStruct((_N, _D), jnp.float32),
)


def _tc_fin_body(acc_ref, g_ref, dinv_ref, b_ref, out_ref):
    out_ref[...] = (dinv_ref[...] * (acc_ref[0, :_N] + acc_ref[1, :_N] + g_ref[...])
                    + b_ref[...])


_tc_fin = pl.pallas_call(
    _tc_fin_body,
    out_shape=jax.ShapeDtypeStruct((_N, _D), jnp.float32),
)


def kernel(x, edge_index, W1, b1, gamma, beta, W2, b2):
    src = edge_index[0].reshape(_NTILES, _NCH, _CH)
    dst = edge_index[1].reshape(_NTILES, _NCH, _CH)
    dst16 = edge_index[1].reshape(_NTILES, _EPT // 16, 16)
    zeros128 = jnp.zeros((_NP, _D), jnp.float32)
    b1r = b1.reshape(1, _D)
    b2r = b2.reshape(1, _D)
    gammar = gamma.reshape(1, _D)
    betar = beta.reshape(1, _D)

    deg_raw = _sc_deg(dst16)
    h1 = _tc_h(x, W1)
    g1, dinv = _tc_scale(h1, deg_raw)
    acc1 = _sc_edge(g1, src, dst, zeros128)
    g2 = _tc_mid(acc1, g1, dinv, b1r, gammar, betar, W2)
    acc2 = _sc_edge(g2, src, dst, zeros128)
    out = _tc_fin(acc2, g2, dinv, b2r)
    return out


# speedup vs baseline: 24.6872x; 1.2394x over previous
import dataclasses
import functools

import jax
import jax.numpy as jnp
from jax import lax
from jax.experimental import pallas as pl
from jax.experimental.pallas import tpu as pltpu
from jax.experimental.pallas import tpu_sc as plsc

_N = 10000
_E = 320000
_D = 128
_NTILES = 32
_EPT = _E // _NTILES
_CH = 100
_EPTF = _E // 16
_NCH = _EPTF // _CH
_HD = _D // 2
_NP = 10240
_RPT = _NP // 16

_mesh = plsc.VectorSubcoreMesh(core_axis_name="c", subcore_axis_name="s")

_sc_params = pltpu.CompilerParams()
if "needs_layout_passes" in pltpu.CompilerParams.__dataclass_fields__:
    _sc_params = dataclasses.replace(_sc_params, needs_layout_passes=False)
_sc_edge_params = dataclasses.replace(_sc_params, use_tc_tiling_on_sc=False)


@functools.partial(
    pl.kernel,
    out_type=jax.ShapeDtypeStruct((_NTILES, _NP), jnp.float32),
    mesh=_mesh,
    scratch_types=[
        pltpu.VMEM((_EPT // 16, 16), jnp.int32),
        pltpu.VMEM((_NP,), jnp.float32),
    ],
    compiler_params=_sc_params,
)
def _sc_deg(dst_hbm, out_hbm, dst_v, hist_v):
    c = lax.axis_index("c")
    s = lax.axis_index("s")
    w = c * 16 + s
    pltpu.sync_copy(dst_hbm.at[w], dst_v)
    zero16 = jnp.zeros((16,), jnp.float32)
    one16 = jnp.full((16,), 1.0, jnp.float32)

    @pl.loop(0, _NP // 16)
    def _(i):
        hist_v[pl.ds(pl.multiple_of(i * 16, 16), 16)] = zero16

    @pl.loop(0, _EPT // 16)
    def _(i):
        plsc.addupdate_scatter(hist_v, [dst_v[i]], one16)

    pltpu.sync_copy(hist_v, out_hbm.at[w])


@functools.partial(
    pl.kernel,
    out_type=jax.ShapeDtypeStruct((2, _NP, _HD), jnp.float32),
    mesh=_mesh,
    scratch_types=[
        pltpu.VMEM((_NCH, _CH), jnp.int32),
        pltpu.VMEM((_NCH, _CH), jnp.int32),
        pltpu.VMEM((_CH, _HD), jnp.float32),
        pltpu.VMEM((_CH, _HD), jnp.float32),
        pltpu.VMEM_SHARED((_NP, _HD), jnp.float32),
        pltpu.SemaphoreType.DMA,
        pltpu.SemaphoreType.DMA,
    ],
    compiler_params=_sc_edge_params,
)
def _sc_edge(g_hbm, src_hbm, dst_hbm, zeros_hbm, out_hbm,
             src_v, dst_v, rows0_v, rows1_v, acc_sh, sem0, sem1):
    c = lax.axis_index("c")
    s = lax.axis_index("s")
    w = c * 16 + s
    pltpu.sync_copy(src_hbm.at[w], src_v)
    pltpu.sync_copy(dst_hbm.at[s], dst_v)
    row0 = s * _RPT
    pltpu.sync_copy(zeros_hbm.at[pl.ds(row0, _RPT)], acc_sh.at[pl.ds(row0, _RPT)])
    plsc.subcore_barrier()

    pltpu.async_copy(g_hbm.at[src_v.at[0]], rows0_v, sem0)

    @pl.loop(0, _NCH - 2, step=2)
    def _(j):
        pltpu.async_copy(g_hbm.at[src_v.at[j + 1]], rows1_v, sem1)
        pltpu.make_async_copy(g_hbm.at[src_v.at[j]], rows0_v, sem0).wait()
        pltpu.sync_copy(rows0_v, acc_sh.at[dst_v.at[j]], add=True)
        pltpu.async_copy(g_hbm.at[src_v.at[j + 2]], rows0_v, sem0)
        pltpu.make_async_copy(g_hbm.at[src_v.at[j + 1]], rows1_v, sem1).wait()
        pltpu.sync_copy(rows1_v, acc_sh.at[dst_v.at[j + 1]], add=True)

    _last = _NCH - 2
    pltpu.async_copy(g_hbm.at[src_v.at[_last + 1]], rows1_v, sem1)
    pltpu.make_async_copy(g_hbm.at[src_v.at[_last]], rows0_v, sem0).wait()
    pltpu.sync_copy(rows0_v, acc_sh.at[dst_v.at[_last]], add=True)
    pltpu.make_async_copy(g_hbm.at[src_v.at[_last + 1]], rows1_v, sem1).wait()
    pltpu.sync_copy(rows1_v, acc_sh.at[dst_v.at[_last + 1]], add=True)

    plsc.subcore_barrier()
    pltpu.sync_copy(acc_sh.at[pl.ds(row0, _RPT)], out_hbm.at[c, pl.ds(row0, _RPT)])


def _tc_h_body(x_ref, w_ref, h_ref):
    h_ref[...] = jnp.dot(x_ref[...], w_ref[...],
                         preferred_element_type=jnp.float32)


_tc_h = pl.pallas_call(
    _tc_h_body,
    out_shape=jax.ShapeDtypeStruct((_N, _D), jnp.float32),
)


def _tc_scale_body(h_ref, dr_ref, g_ref, dinv_ref):
    drt = jnp.transpose(dr_ref[...])
    deg = jnp.sum(drt[:_N], axis=1, keepdims=True) + 1.0
    dinv = lax.rsqrt(deg)
    dinv_ref[...] = dinv
    g = h_ref[...] * dinv
    g_ref[0:_N] = g[:, 0:_HD]
    g_ref[_N:2 * _N] = g[:, _HD:_D]


_tc_scale = pl.pallas_call(
    _tc_scale_body,
    out_shape=[
        jax.ShapeDtypeStruct((2 * _N, _HD), jnp.float32),
        jax.ShapeDtypeStruct((_N, 1), jnp.float32),
    ],
)


def _unsplit(acc_ref, g_ref):
    s_plus_g = jnp.concatenate(
        [acc_ref[0, :_N] + g_ref[0:_N], acc_ref[1, :_N] + g_ref[_N:2 * _N]],
        axis=1)
    return s_plus_g


def _tc_mid_body(acc_ref, g_ref, dinv_ref, b_ref, gam_ref, bet_ref, w_ref,
                 out_ref):
    dinv = dinv_ref[...]
    z = dinv * _unsplit(acc_ref, g_ref) + b_ref[...]
    mean = jnp.mean(z, axis=0, keepdims=True)
    var = jnp.mean((z - mean) ** 2, axis=0, keepdims=True)
    z = (z - mean) * lax.rsqrt(var + 1e-5) * gam_ref[...] + bet_ref[...]
    z = jnp.maximum(z, 0.0)
    h2 = jnp.dot(z, w_ref[...], preferred_element_type=jnp.float32)
    g2 = h2 * dinv
    out_ref[0:_N] = g2[:, 0:_HD]
    out_ref[_N:2 * _N] = g2[:, _HD:_D]


_tc_mid = pl.pallas_call(
    _tc_mid_body,
    out_shape=jax.ShapeDtypeStruct((2 * _N, _HD), jnp.float32),
)


def _tc_fin_body(acc_ref, g_ref, dinv_ref, b_ref, out_ref):
    out_ref[...] = dinv_ref[...] * _unsplit(acc_ref, g_ref) + b_ref[...]


_tc_fin = pl.pallas_call(
    _tc_fin_body,
    out_shape=jax.ShapeDtypeStruct((_N, _D), jnp.float32),
)


def kernel(x, edge_index, W1, b1, gamma, beta, W2, b2):
    src16 = edge_index[0].reshape(16, _NCH, _CH)
    srcadj = jnp.stack([src16, src16 + _N]).reshape(_NTILES, _NCH, _CH)
    dst = edge_index[1].reshape(16, _NCH, _CH)
    dst16 = edge_index[1].reshape(_NTILES, _EPT // 16, 16)
    zeros64 = jnp.zeros((_NP, _HD), jnp.float32)
    b1r = b1.reshape(1, _D)
    b2r = b2.reshape(1, _D)
    gammar = gamma.reshape(1, _D)
    betar = beta.reshape(1, _D)

    deg_raw = _sc_deg(dst16)
    h1 = _tc_h(x, W1)
    g1, dinv = _tc_scale(h1, deg_raw)
    acc1 = _sc_edge(g1, srcadj, dst, zeros64)
    g2 = _tc_mid(acc1, g1, dinv, b1r, gammar, betar, W2)
    acc2 = _sc_edge(g2, srcadj, dst, zeros64)
    out = _tc_fin(acc2, g2, dinv, b2r)
    return out
